# Initial kernel scaffold; baseline (speedup 1.0000x reference)
import jax
import jax.numpy as jnp
from jax.experimental import pallas as pl


def kernel(x, pos, edge_index, batch, W1, b1, W2, b2, W3, b3, W4, b4, W5, b5, WL1, bL1, WL2, bL2):
    raise NotImplementedError("write your pallas kernel here")


# speedup vs baseline: 2.2127x; 2.2127x over previous
import functools

import jax
import jax.numpy as jnp
from jax import lax
from jax.experimental import pallas as pl
from jax.experimental.pallas import tpu as pltpu
from jax.experimental.pallas import tpu_sc as plsc

N = 50000
E = 800000
B = 16
C1 = 32
C2 = 64
POOL_OUT = 8
MAX_DIM = 1.0
GRID1 = 0.03125
GRID2 = 0.0625
NCLS = 100

NPAD = 50048
SENT = NPAD - 1
ECHUNK = 8192
EPAD = 819200
PPAD = 57344
GPAD = 1152
NWORK = 32

_mesh = plsc.VectorSubcoreMesh(core_axis_name="c", subcore_axis_name="s",
                               num_cores=2, num_subcores=16)


def _make_segmax(nchan, epad, spad):
  nchunks = epad // ECHUNK
  npass = nchan // NWORK if nchan >= NWORK else 1
  ngrp = ECHUNK // 16

  @functools.partial(
      pl.kernel,
      out_type=jax.ShapeDtypeStruct((nchan, spad), jnp.float32),
      mesh=_mesh,
      compiler_params=pltpu.CompilerParams(needs_layout_passes=False),
      scratch_types=[
          pltpu.VMEM((NPAD,), jnp.float32),
          pltpu.VMEM((spad,), jnp.float32),
          pltpu.VMEM((ECHUNK,), jnp.int32),
          pltpu.VMEM((ECHUNK,), jnp.int32),
      ],
  )
  def k(tab_hbm, src_hbm, dst_hbm, vb_hbm, out_hbm, tab_v, acc_v, s_v, d_v):
    wid = lax.axis_index("s") * 2 + lax.axis_index("c")
    for p in range(npass):
      ch = p * NWORK + wid
      if True:
        pltpu.sync_copy(tab_hbm.at[ch], tab_v)

        def init_body(i, _):
          acc_v[pl.ds(i * 16, 16)] = jnp.full((16,), -jnp.inf, jnp.float32)
          return 0
        lax.fori_loop(0, spad // 16, init_body, 0)

        def chunk_body(cidx, _):
          pltpu.sync_copy(src_hbm.at[pl.ds(cidx * ECHUNK, ECHUNK)], s_v)
          pltpu.sync_copy(dst_hbm.at[pl.ds(cidx * ECHUNK, ECHUNK)], d_v)

          def grp_body(g, _):
            s = s_v[pl.ds(g * 16, 16)]
            d = d_v[pl.ds(g * 16, 16)]
            val = plsc.load_gather(tab_v, [s])
            cur = plsc.load_gather(acc_v, [d])
            plsc.store_scatter(acc_v, [d], jnp.maximum(cur, val),
                               mask=val > cur)
            act = val > plsc.load_gather(acc_v, [d])

            def retry_cond(carry):
              a, it = carry
              return jnp.logical_and(jnp.any(a), it < 16)

            def retry_body(carry):
              a, it = carry
              cur2 = plsc.load_gather(acc_v, [d])
              plsc.store_scatter(acc_v, [d], jnp.maximum(cur2, val),
                                 mask=jnp.logical_and(a, val > cur2))
              a2 = val > plsc.load_gather(acc_v, [d])
              return a2, it + 1

            lax.while_loop(retry_cond, retry_body, (act, jnp.int32(0)))
            return 0
          lax.fori_loop(0, ngrp, grp_body, 0)
          return 0
        lax.fori_loop(0, nchunks, chunk_body, 0)

        pltpu.sync_copy(vb_hbm.at[ch], tab_v.at[pl.ds(0, spad)])

        def epi_body(i, _):
          a = acc_v[pl.ds(i * 16, 16)]
          vb = tab_v[pl.ds(i * 16, 16)]
          acc_v[pl.ds(i * 16, 16)] = jnp.maximum(a - vb, 0.0)
          return 0
        lax.fori_loop(0, spad // 16, epi_body, 0)

        pltpu.sync_copy(acc_v, out_hbm.at[ch])
    return
  return k


def _bf16_round_bits(bits):
  m = jnp.bitwise_and(lax.shift_right_logical(bits, 16), 1)
  return jnp.bitwise_and(bits + 0x7FFF + m, jnp.int32(-65536))


ECONV = 4096


def _make_segmax_conv(nchan, epad, spad):
  nchunks = epad // ECONV
  npass = nchan // NWORK
  ngrp = ECONV // 16

  @functools.partial(
      pl.kernel,
      out_type=jax.ShapeDtypeStruct((nchan, spad), jnp.float32),
      mesh=_mesh,
      compiler_params=pltpu.CompilerParams(needs_layout_passes=False),
      scratch_types=[
          pltpu.VMEM((NPAD,), jnp.float32),
          pltpu.VMEM((spad,), jnp.float32),
          pltpu.VMEM((ECONV,), jnp.int32),
          pltpu.VMEM((ECONV,), jnp.int32),
          pltpu.VMEM((ECONV,), jnp.int32),
          pltpu.VMEM((128,), jnp.float32),
          pltpu.VMEM((128,), jnp.float32),
      ],
  )
  def k(tab_hbm, src_hbm, dst_hbm, rel_hbm, w0_hbm, w1_hbm, vb_hbm, out_hbm,
        tab_v, acc_v, s_v, d_v, r_v, w0_v, w1_v):
    wid = lax.axis_index("s") * 2 + lax.axis_index("c")
    for p in range(npass):
      ch = p * NWORK + wid
      pltpu.sync_copy(tab_hbm.at[ch], tab_v)
      pltpu.sync_copy(w0_hbm.at[ch], w0_v)
      pltpu.sync_copy(w1_hbm.at[ch], w1_v)

      def init_body(i, _):
        acc_v[pl.ds(i * 16, 16)] = jnp.full((16,), -jnp.inf, jnp.float32)
        return 0
      lax.fori_loop(0, spad // 16, init_body, 0)

      def chunk_body(cidx, _):
        pltpu.sync_copy(src_hbm.at[pl.ds(cidx * ECONV, ECONV)], s_v)
        pltpu.sync_copy(dst_hbm.at[pl.ds(cidx * ECONV, ECONV)], d_v)
        pltpu.sync_copy(rel_hbm.at[pl.ds(cidx * ECONV, ECONV)], r_v)
        w0 = w0_v[pl.ds(0, 16)]
        w1 = w1_v[pl.ds(0, 16)]

        def grp_body(g, _):
          s = s_v[pl.ds(g * 16, 16)]
          d = d_v[pl.ds(g * 16, 16)]
          rp = r_v[pl.ds(g * 16, 16)]
          r0 = plsc.bitcast(jnp.bitwise_and(rp, jnp.int32(-65536)),
                            jnp.float32)
          r1 = plsc.bitcast(lax.shift_left(rp, 16), jnp.float32)
          val = plsc.load_gather(tab_v, [s]) + r0 * w0 + r1 * w1
          cur = plsc.load_gather(acc_v, [d])
          plsc.store_scatter(acc_v, [d], jnp.maximum(cur, val),
                             mask=val > cur)
          act = val > plsc.load_gather(acc_v, [d])

          def retry_cond(carry):
            a, it = carry
            return jnp.logical_and(jnp.any(a), it < 16)

          def retry_body(carry):
            a, it = carry
            cur2 = plsc.load_gather(acc_v, [d])
            plsc.store_scatter(acc_v, [d], jnp.maximum(cur2, val),
                               mask=jnp.logical_and(a, val > cur2))
            a2 = val > plsc.load_gather(acc_v, [d])
            return a2, it + 1

          lax.while_loop(retry_cond, retry_body, (act, jnp.int32(0)))
          return 0
        lax.fori_loop(0, ngrp, grp_body, 0)
        return 0
      lax.fori_loop(0, nchunks, chunk_body, 0)

      pltpu.sync_copy(vb_hbm.at[ch], tab_v.at[pl.ds(0, spad)])

      def epi_body(i, _):
        a = acc_v[pl.ds(i * 16, 16)]
        vb = tab_v[pl.ds(i * 16, 16)]
        acc_v[pl.ds(i * 16, 16)] = jnp.maximum(a - vb, 0.0)
        return 0
      lax.fori_loop(0, spad // 16, epi_body, 0)

      pltpu.sync_copy(acc_v, out_hbm.at[ch])
    return
  return k


RELCHUNK = 3200


@functools.partial(
    pl.kernel,
    out_type=jax.ShapeDtypeStruct((EPAD,), jnp.int32),
    mesh=_mesh,
    compiler_params=pltpu.CompilerParams(needs_layout_passes=False),
    scratch_types=[
        pltpu.VMEM((NPAD,), jnp.float32),
        pltpu.VMEM((NPAD,), jnp.float32),
        pltpu.VMEM((RELCHUNK,), jnp.int32),
        pltpu.VMEM((RELCHUNK,), jnp.int32),
        pltpu.VMEM((RELCHUNK,), jnp.int32),
    ],
)
def _relpack(pos01_hbm, src_hbm, dst_hbm, out_hbm, p0_v, p1_v, s_v, d_v, o_v):
  wid = lax.axis_index("s") * 2 + lax.axis_index("c")
  per = EPAD // NWORK
  base = wid * per
  pltpu.sync_copy(pos01_hbm.at[0], p0_v)
  pltpu.sync_copy(pos01_hbm.at[1], p1_v)

  def chunk_body(cidx, _):
    off = base + cidx * RELCHUNK
    pltpu.sync_copy(src_hbm.at[pl.ds(off, RELCHUNK)], s_v)
    pltpu.sync_copy(dst_hbm.at[pl.ds(off, RELCHUNK)], d_v)

    def grp_body(g, _):
      s = s_v[pl.ds(g * 16, 16)]
      d = d_v[pl.ds(g * 16, 16)]
      r0 = plsc.load_gather(p0_v, [s]) - plsc.load_gather(p0_v, [d])
      r1 = plsc.load_gather(p1_v, [s]) - plsc.load_gather(p1_v, [d])
      b0 = _bf16_round_bits(plsc.bitcast(r0, jnp.int32))
      b1 = _bf16_round_bits(plsc.bitcast(r1, jnp.int32))
      o_v[pl.ds(g * 16, 16)] = jnp.bitwise_or(
          b0, lax.shift_right_logical(b1, 16))
      return 0
    lax.fori_loop(0, RELCHUNK // 16, grp_body, 0)
    pltpu.sync_copy(o_v, out_hbm.at[pl.ds(off, RELCHUNK)])
    return 0
  lax.fori_loop(0, per // RELCHUNK, chunk_body, 0)


_segmax_conv32 = _make_segmax_conv(C1, EPAD, NPAD)
_segmax_conv64 = _make_segmax_conv(C2, EPAD, NPAD)
_segmax_pool32 = _make_segmax(C1, PPAD, NPAD)
_segmax_pool64 = _make_segmax(C2, PPAD, NPAD)
_segmax_gridout = _make_segmax(C2, PPAD, GPAD)

RCHUNK = 6400
RTOT = 2 * EPAD


@functools.partial(
    pl.kernel,
    out_type=jax.ShapeDtypeStruct((RTOT,), jnp.int32),
    mesh=_mesh,
    compiler_params=pltpu.CompilerParams(needs_layout_passes=False),
    scratch_types=[
        pltpu.VMEM((NPAD,), jnp.int32),
        pltpu.VMEM((RCHUNK,), jnp.int32),
        pltpu.VMEM((RCHUNK,), jnp.int32),
    ],
)
def _remap_edges(inv_hbm, ei_hbm, out_hbm, inv_v, in_v, out_v):
  wid = lax.axis_index("s") * 2 + lax.axis_index("c")
  per = RTOT // NWORK
  base = wid * per
  pltpu.sync_copy(inv_hbm, inv_v)

  def chunk_body(cidx, _):
    off = base + cidx * RCHUNK
    pltpu.sync_copy(ei_hbm.at[pl.ds(off, RCHUNK)], in_v)

    def grp_body(g, _):
      idx = in_v[pl.ds(g * 16, 16)]
      out_v[pl.ds(g * 16, 16)] = plsc.load_gather(inv_v, [idx])
      return 0
    lax.fori_loop(0, RCHUNK // 16, grp_body, 0)
    pltpu.sync_copy(out_v, out_hbm.at[pl.ds(off, RCHUNK)])
    return 0
  lax.fori_loop(0, per // RCHUNK, chunk_body, 0)


def _mm_body(a_ref, b_ref, o_ref):
  o_ref[...] = jnp.dot(a_ref[...], b_ref[...],
                       preferred_element_type=jnp.float32)


def _node_mm(a, bmat):
  k = a.shape[1]
  m = bmat.shape[1]
  bn = NPAD // 16
  return pl.pallas_call(
      _mm_body,
      grid=(16,),
      in_specs=[
          pl.BlockSpec((bn, k), lambda i: (i, 0)),
          pl.BlockSpec((k, m), lambda i: (0, 0)),
      ],
      out_specs=pl.BlockSpec((bn, m), lambda i: (i, 0)),
      out_shape=jax.ShapeDtypeStruct((NPAD, m), jnp.float32),
  )(a, bmat)


def _head_body(g_ref, w1_ref, b1_ref, w2_ref, b2_ref, o_ref):
  z = jnp.dot(g_ref[...], w1_ref[...], preferred_element_type=jnp.float32)
  z = jnp.maximum(z + b1_ref[...], 0.0)
  z = jnp.dot(z, w2_ref[...], preferred_element_type=jnp.float32)
  z = z + b2_ref[...]
  z = z - jnp.max(z, axis=-1, keepdims=True)
  o_ref[...] = z - jnp.log(jnp.sum(jnp.exp(z), axis=-1, keepdims=True))


def _head(g, w1, b1, w2p, b2p):
  return pl.pallas_call(
      _head_body,
      out_shape=jax.ShapeDtypeStruct((B, 128), jnp.float32),
  )(g, w1, b1.reshape(1, 128), w2p, b2p.reshape(1, 128))


def _conv_consts(w, b, cout):
  rnd = lambda a: a.astype(jnp.bfloat16).astype(jnp.float32)
  w0t = jnp.broadcast_to(rnd(w[-2])[:, None], (cout, 128))
  w1t = jnp.broadcast_to(rnd(w[-1])[:, None], (cout, 128))
  vbt = jnp.broadcast_to(-b[:, None], (cout, NPAD))
  return w0t, w1t, vbt


def _u_mm(x_pad, wx):
  kpad = ((wx.shape[0] + 7) // 8) * 8
  wxp = jnp.pad(wx, ((0, kpad - wx.shape[0]), (0, 0)))
  xp = jnp.pad(x_pad, ((0, 0), (0, kpad - x_pad.shape[1])))
  return _node_mm(xp, wxp)


def _pad_edges(src, dst, epad):
  ne = src.shape[0]
  src = jnp.concatenate([src, jnp.zeros((epad - ne,), jnp.int32)])
  dst = jnp.concatenate([dst, jnp.full((epad - ne,), SENT, jnp.int32)])
  return src, dst


def _cluster_ids(pos, batch, grid, mask):
  nv = int(round(MAX_DIM / grid)) + 1
  v = jnp.clip(jnp.floor(pos / grid).astype(jnp.int32), 0, nv - 1)
  key = ((batch * nv + v[:, 0]) * nv + v[:, 1]) * nv + v[:, 2]
  sentinel = B * nv * nv * nv
  key = jnp.where(mask, key, sentinel)
  clusters, inv = jnp.unique(key, return_inverse=True, size=key.shape[0],
                             fill_value=sentinel)
  inv = inv.reshape(-1).astype(jnp.int32)
  new_batch = (clusters // (nv * nv * nv)).astype(jnp.int32)
  new_mask = clusters < sentinel
  return inv, new_batch, new_mask


_POOL_SRC = None


def _pool_src():
  global _POOL_SRC
  if _POOL_SRC is None:
    _POOL_SRC = jnp.concatenate(
        [jnp.arange(N, dtype=jnp.int32), jnp.zeros((PPAD - N,), jnp.int32)])
  return _POOL_SRC


def kernel(x, pos, edge_index, batch, W1, b1, W2, b2, W3, b3, W4, b4, W5, b5,
           WL1, bL1, WL2, bL2):
  pos2 = pos[:, :2]
  zpadN = lambda a: jnp.pad(a, ((0, NPAD - N), (0, 0)))
  zpadT = lambda a: jnp.pad(a.T, ((0, 0), (0, NPAD - N)))
  zeros_vb32 = jnp.zeros((C1, NPAD), jnp.float32)
  zeros_vb64 = jnp.zeros((C2, NPAD), jnp.float32)

  src0, dst0 = _pad_edges(edge_index[0], edge_index[1], EPAD)

  rel0 = _relpack(zpadT(pos2), src0, dst0)
  u0 = _u_mm(zpadN(x), W1[:1])
  w0t0, w1t0, vb0 = _conv_consts(W1, b1, C1)
  h1_t = _segmax_conv32(u0.T.copy(), src0, dst0, rel0, w0t0, w1t0,
                        vb0)

  mask0 = jnp.ones((N,), dtype=bool)
  inv1, bt1, mk1 = _cluster_ids(pos, batch, GRID1, mask0)
  psrc = _pool_src()
  pdst1 = jnp.concatenate([inv1, jnp.full((PPAD - N,), SENT, jnp.int32)])
  h1p_t = _segmax_pool32(h1_t, psrc, pdst1, zeros_vb32)
  cnt1 = jax.ops.segment_sum(jnp.ones((N, 1), jnp.float32), inv1,
                             num_segments=N)
  pos1 = jax.ops.segment_sum(pos, inv1, num_segments=N) / jnp.maximum(cnt1, 1.0)
  inv1p = jnp.concatenate(
      [inv1, jnp.full((NPAD - N,), SENT, jnp.int32)])
  ei1 = _remap_edges(inv1p, jnp.concatenate([src0, dst0]))
  src1, dst1 = ei1[:EPAD], ei1[EPAD:]

  p12 = pos1[:, :2]
  rel1 = _relpack(zpadT(p12), src1, dst1)
  h1p = h1p_t.T[:N]
  u1 = _u_mm(zpadN(h1p), W2[:C1])
  w0t1, w1t1, vb1 = _conv_consts(W2, b2, C2)
  h2_t = _segmax_conv64(u1.T.copy(), src1, dst1, rel1, w0t1, w1t1,
                        vb1)

  h2 = h2_t.T[:N]
  u2 = _u_mm(zpadN(h2), W3[:C2])
  w0t2, w1t2, vb2 = _conv_consts(W3, b3, C2)
  h3_t = _segmax_conv64(u2.T.copy(), src1, dst1, rel1, w0t2, w1t2, vb2)

  inv2, bt2, mk2 = _cluster_ids(pos1, bt1, GRID2, mk1)
  pdst2 = jnp.concatenate([inv2, jnp.full((PPAD - N,), SENT, jnp.int32)])
  h3p_t = _segmax_pool64(h3_t, psrc, pdst2, zeros_vb64)
  cnt2 = jax.ops.segment_sum(jnp.ones((N, 1), jnp.float32), inv2,
                             num_segments=N)
  pos2b = jax.ops.segment_sum(pos1, inv2, num_segments=N) / jnp.maximum(
      cnt2, 1.0)
  inv2p = jnp.concatenate([inv2, jnp.full((NPAD - N,), SENT, jnp.int32)])
  ei2 = _remap_edges(inv2p, ei1)
  src2, dst2 = ei2[:EPAD], ei2[EPAD:]

  p22 = pos2b[:, :2]
  rel2 = _relpack(zpadT(p22), src2, dst2)
  h3p = h3p_t.T[:N]
  u3 = _u_mm(zpadN(h3p), W4[:C2])
  w0t3, w1t3, vb3 = _conv_consts(W4, b4, C2)
  h4_t = _segmax_conv64(u3.T.copy(), src2, dst2, rel2, w0t3, w1t3, vb3)

  h4 = h4_t.T[:N]
  u4 = _u_mm(zpadN(h4), W5[:C2])
  w0t4, w1t4, vb4 = _conv_consts(W5, b5, C2)
  h5_t = _segmax_conv64(u4.T.copy(), src2, dst2, rel2, w0t4, w1t4, vb4)

  c = jnp.clip(jnp.floor(p22 / MAX_DIM * POOL_OUT).astype(jnp.int32), 0,
               POOL_OUT - 1)
  gidx = bt2 * POOL_OUT * POOL_OUT + c[:, 0] * POOL_OUT + c[:, 1]
  gidx = jnp.where(mk2, gidx, B * POOL_OUT * POOL_OUT)
  gdst = jnp.concatenate([gidx, jnp.full((PPAD - N,), GPAD - 1, jnp.int32)])
  g_t = _segmax_gridout(h5_t, psrc, gdst,
                        jnp.zeros((C2, GPAD), jnp.float32))
  g = g_t.T[:B * POOL_OUT * POOL_OUT].reshape(B, POOL_OUT * POOL_OUT * C2)

  w2p = jnp.pad(WL2, ((0, 0), (0, 128 - NCLS)))
  b2p = jnp.concatenate(
      [bL2, jnp.full((128 - NCLS,), -1e30, jnp.float32)])
  out = _head(g, WL1, bL1, w2p, b2p)
  return out[:, :NCLS]


# speedup vs baseline: 2.3526x; 1.0632x over previous
import functools

import jax
import jax.numpy as jnp
from jax import lax
from jax.experimental import pallas as pl
from jax.experimental.pallas import tpu as pltpu
from jax.experimental.pallas import tpu_sc as plsc

N = 50000
E = 800000
B = 16
C1 = 32
C2 = 64
POOL_OUT = 8
MAX_DIM = 1.0
GRID1 = 0.03125
GRID2 = 0.0625
NCLS = 100

NPAD = 50048
SENT = NPAD - 1
ECHUNK = 8192
EPAD = 819200
PPAD = 57344
GPAD = 1152
NWORK = 32

_mesh = plsc.VectorSubcoreMesh(core_axis_name="c", subcore_axis_name="s",
                               num_cores=2, num_subcores=16)


def _make_segmax(nchan, epad, spad):
  nchunks = epad // ECHUNK
  npass = nchan // NWORK if nchan >= NWORK else 1
  ngrp = ECHUNK // 16

  @functools.partial(
      pl.kernel,
      out_type=jax.ShapeDtypeStruct((nchan, spad), jnp.float32),
      mesh=_mesh,
      compiler_params=pltpu.CompilerParams(needs_layout_passes=False),
      scratch_types=[
          pltpu.VMEM((NPAD,), jnp.float32),
          pltpu.VMEM((spad,), jnp.float32),
          pltpu.VMEM((ECHUNK,), jnp.int32),
          pltpu.VMEM((ECHUNK,), jnp.int32),
      ],
  )
  def k(tab_hbm, src_hbm, dst_hbm, vb_hbm, out_hbm, tab_v, acc_v, s_v, d_v):
    wid = lax.axis_index("s") * 2 + lax.axis_index("c")
    for p in range(npass):
      ch = p * NWORK + wid
      if True:
        pltpu.sync_copy(tab_hbm.at[ch], tab_v)

        def init_body(i, _):
          acc_v[pl.ds(i * 16, 16)] = jnp.full((16,), -jnp.inf, jnp.float32)
          return 0
        lax.fori_loop(0, spad // 16, init_body, 0)

        def chunk_body(cidx, _):
          pltpu.sync_copy(src_hbm.at[pl.ds(cidx * ECHUNK, ECHUNK)], s_v)
          pltpu.sync_copy(dst_hbm.at[pl.ds(cidx * ECHUNK, ECHUNK)], d_v)

          def grp_body(g, _):
            s = s_v[pl.ds(g * 16, 16)]
            d = d_v[pl.ds(g * 16, 16)]
            val = plsc.load_gather(tab_v, [s])
            cur = plsc.load_gather(acc_v, [d])
            plsc.store_scatter(acc_v, [d], jnp.maximum(cur, val),
                               mask=val > cur)
            act = val > plsc.load_gather(acc_v, [d])

            def retry_cond(carry):
              a, it = carry
              return jnp.logical_and(jnp.any(a), it < 16)

            def retry_body(carry):
              a, it = carry
              cur2 = plsc.load_gather(acc_v, [d])
              plsc.store_scatter(acc_v, [d], jnp.maximum(cur2, val),
                                 mask=jnp.logical_and(a, val > cur2))
              a2 = val > plsc.load_gather(acc_v, [d])
              return a2, it + 1

            lax.while_loop(retry_cond, retry_body, (act, jnp.int32(0)))
            return 0
          lax.fori_loop(0, ngrp, grp_body, 0)
          return 0
        lax.fori_loop(0, nchunks, chunk_body, 0)

        pltpu.sync_copy(vb_hbm.at[ch], tab_v.at[pl.ds(0, spad)])

        def epi_body(i, _):
          a = acc_v[pl.ds(i * 16, 16)]
          vb = tab_v[pl.ds(i * 16, 16)]
          acc_v[pl.ds(i * 16, 16)] = jnp.maximum(a - vb, 0.0)
          return 0
        lax.fori_loop(0, spad // 16, epi_body, 0)

        pltpu.sync_copy(acc_v, out_hbm.at[ch])
    return
  return k


def _bf16_round_bits(bits):
  m = jnp.bitwise_and(lax.shift_right_logical(bits, 16), 1)
  return jnp.bitwise_and(bits + 0x7FFF + m, jnp.int32(-65536))


ECONV = 8192


def _make_segmax_conv(nchan, epad, spad):
  nchunks = epad // ECONV
  npass = nchan // NWORK
  ngrp = ECONV // 16

  @functools.partial(
      pl.kernel,
      out_type=jax.ShapeDtypeStruct((nchan, spad), jnp.float32),
      mesh=_mesh,
      compiler_params=pltpu.CompilerParams(needs_layout_passes=False),
      scratch_types=[
          pltpu.VMEM((NPAD,), jnp.float32),
          pltpu.VMEM((spad,), jnp.float32),
          pltpu.VMEM((ECONV,), jnp.int32),
          pltpu.VMEM((ECONV,), jnp.int32),
          pltpu.VMEM((ECONV,), jnp.int32),
          pltpu.VMEM((128,), jnp.float32),
          pltpu.VMEM((128,), jnp.float32),
      ],
  )
  def k(tab_hbm, src_hbm, dst_hbm, rel_hbm, w0_hbm, w1_hbm, vb_hbm, out_hbm,
        tab_v, acc_v, s_v, d_v, r_v, w0_v, w1_v):
    wid = lax.axis_index("s") * 2 + lax.axis_index("c")
    for p in range(npass):
      ch = p * NWORK + wid
      pltpu.sync_copy(tab_hbm.at[ch], tab_v)
      pltpu.sync_copy(w0_hbm.at[ch], w0_v)
      pltpu.sync_copy(w1_hbm.at[ch], w1_v)

      def init_body(i, _):
        acc_v[pl.ds(i * 16, 16)] = jnp.full((16,), -jnp.inf, jnp.float32)
        return 0
      lax.fori_loop(0, spad // 16, init_body, 0)

      def chunk_body(cidx, _):
        pltpu.sync_copy(src_hbm.at[pl.ds(cidx * ECONV, ECONV)], s_v)
        pltpu.sync_copy(dst_hbm.at[pl.ds(cidx * ECONV, ECONV)], d_v)
        pltpu.sync_copy(rel_hbm.at[pl.ds(cidx * ECONV, ECONV)], r_v)
        w0 = w0_v[pl.ds(0, 16)]
        w1 = w1_v[pl.ds(0, 16)]

        def grp_body(g, _):
          s = s_v[pl.ds(g * 16, 16)]
          d = d_v[pl.ds(g * 16, 16)]
          rp = r_v[pl.ds(g * 16, 16)]
          r0 = plsc.bitcast(jnp.bitwise_and(rp, jnp.int32(-65536)),
                            jnp.float32)
          r1 = plsc.bitcast(lax.shift_left(rp, 16), jnp.float32)
          val = plsc.load_gather(tab_v, [s]) + r0 * w0 + r1 * w1
          cur = plsc.load_gather(acc_v, [d])
          plsc.store_scatter(acc_v, [d], jnp.maximum(cur, val),
                             mask=val > cur)
          act = val > plsc.load_gather(acc_v, [d])

          def retry_cond(carry):
            a, it = carry
            return jnp.logical_and(jnp.any(a), it < 16)

          def retry_body(carry):
            a, it = carry
            cur2 = plsc.load_gather(acc_v, [d])
            plsc.store_scatter(acc_v, [d], jnp.maximum(cur2, val),
                               mask=jnp.logical_and(a, val > cur2))
            a2 = val > plsc.load_gather(acc_v, [d])
            return a2, it + 1

          lax.while_loop(retry_cond, retry_body, (act, jnp.int32(0)))
          return 0
        lax.fori_loop(0, ngrp, grp_body, 0)
        return 0
      lax.fori_loop(0, nchunks, chunk_body, 0)

      pltpu.sync_copy(vb_hbm.at[ch], tab_v.at[pl.ds(0, spad)])

      def epi_body(i, _):
        a = acc_v[pl.ds(i * 16, 16)]
        vb = tab_v[pl.ds(i * 16, 16)]
        acc_v[pl.ds(i * 16, 16)] = jnp.maximum(a - vb, 0.0)
        return 0
      lax.fori_loop(0, spad // 16, epi_body, 0)

      pltpu.sync_copy(acc_v, out_hbm.at[ch])
    return
  return k


RELCHUNK = 3200


@functools.partial(
    pl.kernel,
    out_type=jax.ShapeDtypeStruct((EPAD,), jnp.int32),
    mesh=_mesh,
    compiler_params=pltpu.CompilerParams(needs_layout_passes=False),
    scratch_types=[
        pltpu.VMEM((NPAD,), jnp.float32),
        pltpu.VMEM((NPAD,), jnp.float32),
        pltpu.VMEM((RELCHUNK,), jnp.int32),
        pltpu.VMEM((RELCHUNK,), jnp.int32),
        pltpu.VMEM((RELCHUNK,), jnp.int32),
    ],
)
def _relpack(pos01_hbm, src_hbm, dst_hbm, out_hbm, p0_v, p1_v, s_v, d_v, o_v):
  wid = lax.axis_index("s") * 2 + lax.axis_index("c")
  per = EPAD // NWORK
  base = wid * per
  pltpu.sync_copy(pos01_hbm.at[0], p0_v)
  pltpu.sync_copy(pos01_hbm.at[1], p1_v)

  def chunk_body(cidx, _):
    off = base + cidx * RELCHUNK
    pltpu.sync_copy(src_hbm.at[pl.ds(off, RELCHUNK)], s_v)
    pltpu.sync_copy(dst_hbm.at[pl.ds(off, RELCHUNK)], d_v)

    def grp_body(g, _):
      s = s_v[pl.ds(g * 16, 16)]
      d = d_v[pl.ds(g * 16, 16)]
      r0 = plsc.load_gather(p0_v, [s]) - plsc.load_gather(p0_v, [d])
      r1 = plsc.load_gather(p1_v, [s]) - plsc.load_gather(p1_v, [d])
      b0 = _bf16_round_bits(plsc.bitcast(r0, jnp.int32))
      b1 = _bf16_round_bits(plsc.bitcast(r1, jnp.int32))
      o_v[pl.ds(g * 16, 16)] = jnp.bitwise_or(
          b0, lax.shift_right_logical(b1, 16))
      return 0
    lax.fori_loop(0, RELCHUNK // 16, grp_body, 0)
    pltpu.sync_copy(o_v, out_hbm.at[pl.ds(off, RELCHUNK)])
    return 0
  lax.fori_loop(0, per // RELCHUNK, chunk_body, 0)


_segmax_conv32 = _make_segmax_conv(C1, EPAD, NPAD)
_segmax_conv64 = _make_segmax_conv(C2, EPAD, NPAD)
_segmax_pool32 = _make_segmax(C1, PPAD, NPAD)
_segmax_pool64 = _make_segmax(C2, PPAD, NPAD)
_segmax_gridout = _make_segmax(C2, PPAD, GPAD)

RCHUNK = 6400
RTOT = 2 * EPAD


@functools.partial(
    pl.kernel,
    out_type=jax.ShapeDtypeStruct((RTOT,), jnp.int32),
    mesh=_mesh,
    compiler_params=pltpu.CompilerParams(needs_layout_passes=False),
    scratch_types=[
        pltpu.VMEM((NPAD,), jnp.int32),
        pltpu.VMEM((RCHUNK,), jnp.int32),
        pltpu.VMEM((RCHUNK,), jnp.int32),
    ],
)
def _remap_edges(inv_hbm, ei_hbm, out_hbm, inv_v, in_v, out_v):
  wid = lax.axis_index("s") * 2 + lax.axis_index("c")
  per = RTOT // NWORK
  base = wid * per
  pltpu.sync_copy(inv_hbm, inv_v)

  def chunk_body(cidx, _):
    off = base + cidx * RCHUNK
    pltpu.sync_copy(ei_hbm.at[pl.ds(off, RCHUNK)], in_v)

    def grp_body(g, _):
      idx = in_v[pl.ds(g * 16, 16)]
      out_v[pl.ds(g * 16, 16)] = plsc.load_gather(inv_v, [idx])
      return 0
    lax.fori_loop(0, RCHUNK // 16, grp_body, 0)
    pltpu.sync_copy(out_v, out_hbm.at[pl.ds(off, RCHUNK)])
    return 0
  lax.fori_loop(0, per // RCHUNK, chunk_body, 0)


def _mm_body(a_ref, b_ref, o_ref):
  o_ref[...] = jnp.dot(a_ref[...], b_ref[...],
                       preferred_element_type=jnp.float32)


def _node_mm(a, bmat):
  k = a.shape[1]
  m = bmat.shape[1]
  bn = NPAD // 16
  return pl.pallas_call(
      _mm_body,
      grid=(16,),
      in_specs=[
          pl.BlockSpec((bn, k), lambda i: (i, 0)),
          pl.BlockSpec((k, m), lambda i: (0, 0)),
      ],
      out_specs=pl.BlockSpec((bn, m), lambda i: (i, 0)),
      out_shape=jax.ShapeDtypeStruct((NPAD, m), jnp.float32),
  )(a, bmat)


def _head_body(g_ref, w1_ref, b1_ref, w2_ref, b2_ref, o_ref):
  z = jnp.dot(g_ref[...], w1_ref[...], preferred_element_type=jnp.float32)
  z = jnp.maximum(z + b1_ref[...], 0.0)
  z = jnp.dot(z, w2_ref[...], preferred_element_type=jnp.float32)
  z = z + b2_ref[...]
  z = z - jnp.max(z, axis=-1, keepdims=True)
  o_ref[...] = z - jnp.log(jnp.sum(jnp.exp(z), axis=-1, keepdims=True))


def _head(g, w1, b1, w2p, b2p):
  return pl.pallas_call(
      _head_body,
      out_shape=jax.ShapeDtypeStruct((B, 128), jnp.float32),
  )(g, w1, b1.reshape(1, 128), w2p, b2p.reshape(1, 128))


def _conv_consts(w, b, cout):
  rnd = lambda a: a.astype(jnp.bfloat16).astype(jnp.float32)
  w0t = jnp.broadcast_to(rnd(w[-2])[:, None], (cout, 128))
  w1t = jnp.broadcast_to(rnd(w[-1])[:, None], (cout, 128))
  vbt = jnp.broadcast_to(-b[:, None], (cout, NPAD))
  return w0t, w1t, vbt


def _u_mm(x_pad, wx):
  kpad = ((wx.shape[0] + 7) // 8) * 8
  wxp = jnp.pad(wx, ((0, kpad - wx.shape[0]), (0, 0)))
  xp = jnp.pad(x_pad, ((0, 0), (0, kpad - x_pad.shape[1])))
  return _node_mm(xp, wxp)


def _pad_edges(src, dst, epad):
  ne = src.shape[0]
  src = jnp.concatenate([src, jnp.zeros((epad - ne,), jnp.int32)])
  dst = jnp.concatenate([dst, jnp.full((epad - ne,), SENT, jnp.int32)])
  return src, dst


def _cluster_ids(pos, batch, grid, mask):
  nv = int(round(MAX_DIM / grid)) + 1
  v = jnp.clip(jnp.floor(pos / grid).astype(jnp.int32), 0, nv - 1)
  key = ((batch * nv + v[:, 0]) * nv + v[:, 1]) * nv + v[:, 2]
  sentinel = B * nv * nv * nv
  key = jnp.where(mask, key, sentinel)
  clusters, inv = jnp.unique(key, return_inverse=True, size=key.shape[0],
                             fill_value=sentinel)
  inv = inv.reshape(-1).astype(jnp.int32)
  new_batch = (clusters // (nv * nv * nv)).astype(jnp.int32)
  new_mask = clusters < sentinel
  return inv, new_batch, new_mask


_POOL_SRC = None


def _pool_src():
  global _POOL_SRC
  if _POOL_SRC is None:
    _POOL_SRC = jnp.concatenate(
        [jnp.arange(N, dtype=jnp.int32), jnp.zeros((PPAD - N,), jnp.int32)])
  return _POOL_SRC


def kernel(x, pos, edge_index, batch, W1, b1, W2, b2, W3, b3, W4, b4, W5, b5,
           WL1, bL1, WL2, bL2):
  pos2 = pos[:, :2]
  zpadN = lambda a: jnp.pad(a, ((0, NPAD - N), (0, 0)))
  zpadT = lambda a: jnp.pad(a.T, ((0, 0), (0, NPAD - N)))
  zeros_vb32 = jnp.zeros((C1, NPAD), jnp.float32)
  zeros_vb64 = jnp.zeros((C2, NPAD), jnp.float32)

  src0, dst0 = _pad_edges(edge_index[0], edge_index[1], EPAD)

  rel0 = _relpack(zpadT(pos2), src0, dst0)
  u0 = _u_mm(zpadN(x), W1[:1])
  w0t0, w1t0, vb0 = _conv_consts(W1, b1, C1)
  h1_t = _segmax_conv32(u0.T.copy(), src0, dst0, rel0, w0t0, w1t0,
                        vb0)

  mask0 = jnp.ones((N,), dtype=bool)
  inv1, bt1, mk1 = _cluster_ids(pos, batch, GRID1, mask0)
  psrc = _pool_src()
  pdst1 = jnp.concatenate([inv1, jnp.full((PPAD - N,), SENT, jnp.int32)])
  h1p_t = _segmax_pool32(h1_t, psrc, pdst1, zeros_vb32)
  cnt1 = jax.ops.segment_sum(jnp.ones((N, 1), jnp.float32), inv1,
                             num_segments=N)
  pos1 = jax.ops.segment_sum(pos, inv1, num_segments=N) / jnp.maximum(cnt1, 1.0)
  inv1p = jnp.concatenate(
      [inv1, jnp.full((NPAD - N,), SENT, jnp.int32)])
  ei1 = _remap_edges(inv1p, jnp.concatenate([src0, dst0]))
  src1, dst1 = ei1[:EPAD], ei1[EPAD:]

  p12 = pos1[:, :2]
  rel1 = _relpack(zpadT(p12), src1, dst1)
  h1p = h1p_t.T[:N]
  u1 = _u_mm(zpadN(h1p), W2[:C1])
  w0t1, w1t1, vb1 = _conv_consts(W2, b2, C2)
  h2_t = _segmax_conv64(u1.T.copy(), src1, dst1, rel1, w0t1, w1t1,
                        vb1)

  h2 = h2_t.T[:N]
  u2 = _u_mm(zpadN(h2), W3[:C2])
  w0t2, w1t2, vb2 = _conv_consts(W3, b3, C2)
  h3_t = _segmax_conv64(u2.T.copy(), src1, dst1, rel1, w0t2, w1t2, vb2)

  inv2, bt2, mk2 = _cluster_ids(pos1, bt1, GRID2, mk1)
  pdst2 = jnp.concatenate([inv2, jnp.full((PPAD - N,), SENT, jnp.int32)])
  h3p_t = _segmax_pool64(h3_t, psrc, pdst2, zeros_vb64)
  cnt2 = jax.ops.segment_sum(jnp.ones((N, 1), jnp.float32), inv2,
                             num_segments=N)
  pos2b = jax.ops.segment_sum(pos1, inv2, num_segments=N) / jnp.maximum(
      cnt2, 1.0)
  inv2p = jnp.concatenate([inv2, jnp.full((NPAD - N,), SENT, jnp.int32)])
  ei2 = _remap_edges(inv2p, ei1)
  src2, dst2 = ei2[:EPAD], ei2[EPAD:]

  p22 = pos2b[:, :2]
  rel2 = _relpack(zpadT(p22), src2, dst2)
  h3p = h3p_t.T[:N]
  u3 = _u_mm(zpadN(h3p), W4[:C2])
  w0t3, w1t3, vb3 = _conv_consts(W4, b4, C2)
  h4_t = _segmax_conv64(u3.T.copy(), src2, dst2, rel2, w0t3, w1t3, vb3)

  h4 = h4_t.T[:N]
  u4 = _u_mm(zpadN(h4), W5[:C2])
  w0t4, w1t4, vb4 = _conv_consts(W5, b5, C2)
  h5_t = _segmax_conv64(u4.T.copy(), src2, dst2, rel2, w0t4, w1t4, vb4)

  c = jnp.clip(jnp.floor(p22 / MAX_DIM * POOL_OUT).astype(jnp.int32), 0,
               POOL_OUT - 1)
  gidx = bt2 * POOL_OUT * POOL_OUT + c[:, 0] * POOL_OUT + c[:, 1]
  gidx = jnp.where(mk2, gidx, B * POOL_OUT * POOL_OUT)
  gdst = jnp.concatenate([gidx, jnp.full((PPAD - N,), GPAD - 1, jnp.int32)])
  g_t = _segmax_gridout(h5_t, psrc, gdst,
                        jnp.zeros((C2, GPAD), jnp.float32))
  g = g_t.T[:B * POOL_OUT * POOL_OUT].reshape(B, POOL_OUT * POOL_OUT * C2)

  w2p = jnp.pad(WL2, ((0, 0), (0, 128 - NCLS)))
  b2p = jnp.concatenate(
      [bL2, jnp.full((128 - NCLS,), -1e30, jnp.float32)])
  out = _head(g, WL1, bL1, w2p, b2p)
  return out[:, :NCLS]
